# baseline (speedup 1.0000x reference)
import functools

import jax
import jax.numpy as jnp
from jax import lax
from jax.experimental import pallas as pl
from jax.experimental.pallas import tpu as pltpu
from jax.experimental.pallas import tpu_sc as plsc

B, L, V, D, H = 16384, 20, 100000, 128, 1024

_INFO = plsc.get_sparse_core_info()
_NC, _NS, _LANES = _INFO.num_cores, _INFO.num_subcores, _INFO.num_lanes
_NW = _NC * _NS
_NSPLIT = 4
_NBATCH = B // _NSPLIT
_RPW = _NBATCH // _NW
_IDX = _RPW * L
_C = 16
_NCHUNK = _RPW // _C
_IPC = _C * L
_GR = 64
_KD = _IPC // _GR
_GD = D // _LANES
_ROWLEN = _RPW + _LANES


def _issue_chunk(c, idx_lmaj, table_hbm, rows_v, sem):
    for l in range(L):
        pltpu.make_async_copy(
            table_hbm.at[idx_lmaj.at[pl.ds(l * _ROWLEN + c * _C, _C)]],
            rows_v.at[pl.ds(l * _C, _C), :], sem).start()


def _wait_chunk(c, idx_lmaj, table_hbm, rows_v, sem):
    for l in range(L):
        pltpu.make_async_copy(
            table_hbm.at[idx_lmaj.at[pl.ds(l * _ROWLEN + c * _C, _C)]],
            rows_v.at[pl.ds(l * _C, _C), :], sem).wait()


def _pool_chunk(c, wrow0, idx_lmaj, rows_v, pooled_v, pooled_hbm):
    def row_body(r, _):
        acc = [jnp.zeros((_LANES,), jnp.float32) for _ in range(_GD)]
        for l in range(L):
            j = l * _C + r
            iv = idx_lmaj[pl.ds(l * _ROWLEN + c * _C + r, _LANES)][0]
            mf = jnp.full((_LANES,),
                          jnp.where(iv != 0, 1.0, 0.0), jnp.float32)
            for g in range(_GD):
                rowg = rows_v[j, pl.ds(_LANES * g, _LANES)]
                acc[g] = acc[g] + rowg * mf
        for g in range(_GD):
            pooled_v[r, pl.ds(_LANES * g, _LANES)] = acc[g] * (1.0 / L)
        return ()

    lax.fori_loop(0, _C, row_body, (), unroll=False)
    pltpu.sync_copy(pooled_v, pooled_hbm.at[pl.ds(wrow0 + c * _C, _C), :])


def _sc_body(split, seq_hbm, item_hbm, table_hbm, pooled_hbm, item_out_hbm,
             idx_lmaj, rows_a, rows_b, pooled_v, sem_a, sem_b):
    wid = lax.axis_index("s") * _NC + lax.axis_index("c")
    wrow0 = wid * _RPW
    grow0 = split * _NBATCH + wrow0

    stages = [
        pltpu.async_copy(seq_hbm.at[pl.ds(l * B + grow0, _RPW)],
                         idx_lmaj.at[pl.ds(l * _ROWLEN, _RPW)], sem_a)
        for l in range(L)
    ]
    for cp in stages:
        cp.wait()

    _issue_chunk(0, idx_lmaj, table_hbm, rows_a, sem_a)

    def pair_body(p, _):
        ca = 2 * p
        cb = 2 * p + 1
        _issue_chunk(cb, idx_lmaj, table_hbm, rows_b, sem_b)
        _wait_chunk(ca, idx_lmaj, table_hbm, rows_a, sem_a)
        _pool_chunk(ca, wrow0, idx_lmaj, rows_a, pooled_v, pooled_hbm)

        @pl.when(p < _NCHUNK // 2 - 1)
        def _():
            _issue_chunk(ca + 2, idx_lmaj, table_hbm, rows_a, sem_a)

        _wait_chunk(cb, idx_lmaj, table_hbm, rows_b, sem_b)
        _pool_chunk(cb, wrow0, idx_lmaj, rows_b, pooled_v, pooled_hbm)
        return ()

    lax.fori_loop(0, _NCHUNK // 2, pair_body, (), unroll=False)

    pltpu.sync_copy(item_hbm.at[pl.ds(wrow0, _RPW)],
                    idx_lmaj.at[pl.ds(0, _RPW)])
    copies = [
        pltpu.async_copy(table_hbm.at[idx_lmaj.at[pl.ds(_GR * k, _GR)]],
                         rows_a.at[pl.ds(_GR * k, _GR), :], sem_a)
        for k in range(_RPW // _GR)
    ]
    for cp in copies:
        cp.wait()
    pltpu.sync_copy(rows_a.at[pl.ds(0, _RPW), :],
                    item_out_hbm.at[pl.ds(wrow0, _RPW), :])


def _sc_gather_pool(split, seq_t_flat, item_split, table):
    kfn = pl.kernel(
        functools.partial(_sc_body, split),
        out_type=(jax.ShapeDtypeStruct((_NBATCH, D), jnp.float32),
                  jax.ShapeDtypeStruct((_NBATCH, D), jnp.float32)),
        mesh=plsc.VectorSubcoreMesh(core_axis_name="c", subcore_axis_name="s"),
        scratch_types=[
            pltpu.VMEM((L * _ROWLEN + _LANES,), jnp.int32),
            pltpu.VMEM((_IPC, D), jnp.float32),
            pltpu.VMEM((_IPC, D), jnp.float32),
            pltpu.VMEM((_C, D), jnp.float32),
            pltpu.SemaphoreType.DMA,
            pltpu.SemaphoreType.DMA,
        ],
    )
    return kfn(seq_t_flat, item_split, table)


_BM = min(2048, B // _NSPLIT)
_NB = _NBATCH // _BM


def _mlp_body(pooled_ref, item_ref, w1_ref, b1_ref, w2_ref, b2_ref, out_ref):
    x = pooled_ref[...].astype(jnp.bfloat16)
    h = jnp.dot(x, w1_ref[...], preferred_element_type=jnp.float32)
    h = jnp.maximum(h + b1_ref[...], 0.0).astype(jnp.bfloat16)
    u = jnp.dot(h, w2_ref[...], preferred_element_type=jnp.float32)
    u = jnp.maximum(u + b2_ref[...], 0.0)
    s = jnp.sum(u * item_ref[...], axis=1)
    out_ref[0, 0, :] = jax.nn.sigmoid(s)


def _tc_mlp(pooled, item_embed, W1b, b1, W2b, b2):
    out = pl.pallas_call(
        _mlp_body,
        grid=(_NB,),
        in_specs=[
            pl.BlockSpec((_BM, D), lambda i: (i, 0)),
            pl.BlockSpec((_BM, D), lambda i: (i, 0)),
            pl.BlockSpec((D, H), lambda i: (0, 0)),
            pl.BlockSpec((1, H), lambda i: (0, 0)),
            pl.BlockSpec((H, D), lambda i: (0, 0)),
            pl.BlockSpec((1, D), lambda i: (0, 0)),
        ],
        out_specs=pl.BlockSpec((1, 1, _BM), lambda i: (i, 0, 0)),
        out_shape=jax.ShapeDtypeStruct((_NB, 1, _BM), jnp.float32),
    )(pooled, item_embed, W1b, b1.reshape(1, H), W2b, b2.reshape(1, D))
    return out.reshape(_NBATCH, 1)


def kernel(seq_inputs, item_inputs, table, W1, b1, W2, b2):
    seq_t_flat = seq_inputs.T.reshape(L * B).astype(jnp.int32)
    item_flat = item_inputs.reshape(B).astype(jnp.int32)
    W1b = W1.astype(jnp.bfloat16)
    W2b = W2.astype(jnp.bfloat16)
    outs = []
    for s in range(_NSPLIT):
        pooled, item_embed = _sc_gather_pool(
            s, seq_t_flat, item_flat[s * _NBATCH:(s + 1) * _NBATCH], table)
        outs.append(_tc_mlp(pooled, item_embed, W1b, b1, W2b, b2))
    return jnp.concatenate(outs, axis=0)


# speedup vs baseline: 1.1344x; 1.1344x over previous
import functools

import jax
import jax.numpy as jnp
from jax import lax
from jax.experimental import pallas as pl
from jax.experimental.pallas import tpu as pltpu
from jax.experimental.pallas import tpu_sc as plsc

B, L, V, D, H = 16384, 20, 100000, 128, 1024

_INFO = plsc.get_sparse_core_info()
_NC, _NS, _LANES = _INFO.num_cores, _INFO.num_subcores, _INFO.num_lanes
_NW = _NC * _NS
_NSPLIT = 2
_NBATCH = B // _NSPLIT
_RPW = _NBATCH // _NW
_IDX = _RPW * L
_C = 16
_NCHUNK = _RPW // _C
_IPC = _C * L
_GR = 64
_KD = _IPC // _GR
_GD = D // _LANES
_ROWLEN = _RPW + _LANES


def _issue_chunk(c, idx_lmaj, table_hbm, rows_v, sem):
    for l in range(L):
        pltpu.make_async_copy(
            table_hbm.at[idx_lmaj.at[pl.ds(l * _ROWLEN + c * _C, _C)]],
            rows_v.at[pl.ds(l * _C, _C), :], sem).start()


def _wait_chunk(c, idx_lmaj, table_hbm, rows_v, sem):
    for l in range(L):
        pltpu.make_async_copy(
            table_hbm.at[idx_lmaj.at[pl.ds(l * _ROWLEN + c * _C, _C)]],
            rows_v.at[pl.ds(l * _C, _C), :], sem).wait()


def _pool_chunk(c, wrow0, idx_lmaj, rows_v, pooled_v, pooled_hbm):
    def row_body(r, _):
        acc = [jnp.zeros((_LANES,), jnp.float32) for _ in range(_GD)]
        for l in range(L):
            j = l * _C + r
            iv = idx_lmaj[pl.ds(l * _ROWLEN + c * _C + r, _LANES)][0]
            mf = jnp.full((_LANES,),
                          jnp.where(iv != 0, 1.0, 0.0), jnp.float32)
            for g in range(_GD):
                rowg = rows_v[j, pl.ds(_LANES * g, _LANES)]
                acc[g] = acc[g] + rowg * mf
        for g in range(_GD):
            pooled_v[r, pl.ds(_LANES * g, _LANES)] = acc[g] * (1.0 / L)
        return ()

    lax.fori_loop(0, _C, row_body, (), unroll=False)
    pltpu.sync_copy(pooled_v, pooled_hbm.at[pl.ds(wrow0 + c * _C, _C), :])


def _sc_body(split, seq_hbm, item_hbm, table_hbm, pooled_hbm, item_out_hbm,
             idx_lmaj, rows_a, rows_b, pooled_v, sem_a, sem_b):
    wid = lax.axis_index("s") * _NC + lax.axis_index("c")
    wrow0 = wid * _RPW
    grow0 = split * _NBATCH + wrow0

    stages = [
        pltpu.async_copy(seq_hbm.at[pl.ds(l * B + grow0, _RPW)],
                         idx_lmaj.at[pl.ds(l * _ROWLEN, _RPW)], sem_a)
        for l in range(L)
    ]
    for cp in stages:
        cp.wait()

    _issue_chunk(0, idx_lmaj, table_hbm, rows_a, sem_a)

    def pair_body(p, _):
        ca = 2 * p
        cb = 2 * p + 1
        _issue_chunk(cb, idx_lmaj, table_hbm, rows_b, sem_b)
        _wait_chunk(ca, idx_lmaj, table_hbm, rows_a, sem_a)
        _pool_chunk(ca, wrow0, idx_lmaj, rows_a, pooled_v, pooled_hbm)

        @pl.when(p < _NCHUNK // 2 - 1)
        def _():
            _issue_chunk(ca + 2, idx_lmaj, table_hbm, rows_a, sem_a)

        _wait_chunk(cb, idx_lmaj, table_hbm, rows_b, sem_b)
        _pool_chunk(cb, wrow0, idx_lmaj, rows_b, pooled_v, pooled_hbm)
        return ()

    lax.fori_loop(0, _NCHUNK // 2, pair_body, (), unroll=False)

    pltpu.sync_copy(item_hbm.at[pl.ds(wrow0, _RPW)],
                    idx_lmaj.at[pl.ds(0, _RPW)])
    copies = [
        pltpu.async_copy(table_hbm.at[idx_lmaj.at[pl.ds(_GR * k, _GR)]],
                         rows_a.at[pl.ds(_GR * k, _GR), :], sem_a)
        for k in range(_RPW // _GR)
    ]
    for cp in copies:
        cp.wait()
    pltpu.sync_copy(rows_a.at[pl.ds(0, _RPW), :],
                    item_out_hbm.at[pl.ds(wrow0, _RPW), :])


def _sc_gather_pool(split, seq_t_flat, item_split, table):
    kfn = pl.kernel(
        functools.partial(_sc_body, split),
        out_type=(jax.ShapeDtypeStruct((_NBATCH, D), jnp.float32),
                  jax.ShapeDtypeStruct((_NBATCH, D), jnp.float32)),
        mesh=plsc.VectorSubcoreMesh(core_axis_name="c", subcore_axis_name="s"),
        scratch_types=[
            pltpu.VMEM((L * _ROWLEN + _LANES,), jnp.int32),
            pltpu.VMEM((_IPC, D), jnp.float32),
            pltpu.VMEM((_IPC, D), jnp.float32),
            pltpu.VMEM((_C, D), jnp.float32),
            pltpu.SemaphoreType.DMA,
            pltpu.SemaphoreType.DMA,
        ],
    )
    return kfn(seq_t_flat, item_split, table)


_BM = min(2048, B // _NSPLIT)
_NB = _NBATCH // _BM


def _mlp_body(pooled_ref, item_ref, w1_ref, b1_ref, w2_ref, b2_ref, out_ref):
    x = pooled_ref[...].astype(jnp.bfloat16)
    h = jnp.dot(x, w1_ref[...], preferred_element_type=jnp.float32)
    h = jnp.maximum(h + b1_ref[...], 0.0).astype(jnp.bfloat16)
    u = jnp.dot(h, w2_ref[...], preferred_element_type=jnp.float32)
    u = jnp.maximum(u + b2_ref[...], 0.0)
    s = jnp.sum(u * item_ref[...], axis=1, keepdims=True)
    out_ref[0] = jax.nn.sigmoid(s)


def _tc_mlp(pooled, item_embed, W1b, b1, W2b, b2):
    out = pl.pallas_call(
        _mlp_body,
        grid=(_NB,),
        in_specs=[
            pl.BlockSpec((_BM, D), lambda i: (i, 0)),
            pl.BlockSpec((_BM, D), lambda i: (i, 0)),
            pl.BlockSpec((D, H), lambda i: (0, 0)),
            pl.BlockSpec((1, H), lambda i: (0, 0)),
            pl.BlockSpec((H, D), lambda i: (0, 0)),
            pl.BlockSpec((1, D), lambda i: (0, 0)),
        ],
        out_specs=pl.BlockSpec((1, _BM, 1), lambda i: (i, 0, 0)),
        out_shape=jax.ShapeDtypeStruct((_NB, _BM, 1), jnp.float32),
    )(pooled, item_embed, W1b, b1.reshape(1, H), W2b, b2.reshape(1, D))
    return out.reshape(_NBATCH, 1)


def kernel(seq_inputs, item_inputs, table, W1, b1, W2, b2):
    seq_t_flat = seq_inputs.T.reshape(L * B).astype(jnp.int32)
    item_flat = item_inputs.reshape(B).astype(jnp.int32)
    W1b = W1.astype(jnp.bfloat16)
    W2b = W2.astype(jnp.bfloat16)
    outs = []
    for s in range(_NSPLIT):
        pooled, item_embed = _sc_gather_pool(
            s, seq_t_flat, item_flat[s * _NBATCH:(s + 1) * _NBATCH], table)
        outs.append(_tc_mlp(pooled, item_embed, W1b, b1, W2b, b2))
    return jnp.concatenate(outs, axis=0)


# speedup vs baseline: 1.1357x; 1.0012x over previous
import functools

import jax
import jax.numpy as jnp
from jax import lax
from jax.experimental import pallas as pl
from jax.experimental.pallas import tpu as pltpu
from jax.experimental.pallas import tpu_sc as plsc

B, L, V, D, H = 16384, 20, 100000, 128, 1024

_INFO = plsc.get_sparse_core_info()
_NC, _NS, _LANES = _INFO.num_cores, _INFO.num_subcores, _INFO.num_lanes
_NW = _NC * _NS
_NSPLIT = 2
_NBATCH = B // _NSPLIT
_RPW = _NBATCH // _NW
_IDX = _RPW * L
_C = 16
_NCHUNK = _RPW // _C
_IPC = _C * L
_GR = 64
_KD = _IPC // _GR
_GD = D // _LANES
_ROWLEN = _RPW + _LANES


def _issue_chunk(c, idx_lmaj, table_hbm, rows_v, sem):
    for l in range(L):
        pltpu.make_async_copy(
            table_hbm.at[idx_lmaj.at[pl.ds(l * _ROWLEN + c * _C, _C)]],
            rows_v.at[pl.ds(l * _C, _C), :], sem).start()


def _wait_chunk(c, idx_lmaj, table_hbm, rows_v, sem):
    for l in range(L):
        pltpu.make_async_copy(
            table_hbm.at[idx_lmaj.at[pl.ds(l * _ROWLEN + c * _C, _C)]],
            rows_v.at[pl.ds(l * _C, _C), :], sem).wait()


def _pool_chunk(c, wrow0, idx_lmaj, rows_v, pooled_v, pooled_hbm):
    def row_body(r, _):
        acc = [jnp.zeros((_LANES,), jnp.float32) for _ in range(_GD)]
        for l in range(L):
            j = l * _C + r
            iv = idx_lmaj[pl.ds(l * _ROWLEN + c * _C + r, _LANES)][0]
            mf = jnp.full((_LANES,),
                          jnp.where(iv != 0, 1.0, 0.0), jnp.float32)
            for g in range(_GD):
                rowg = rows_v[j, pl.ds(_LANES * g, _LANES)]
                acc[g] = acc[g] + rowg * mf
        for g in range(_GD):
            pooled_v[r, pl.ds(_LANES * g, _LANES)] = acc[g] * (1.0 / L)
        return ()

    lax.fori_loop(0, _C, row_body, (), unroll=False)
    pltpu.sync_copy(pooled_v, pooled_hbm.at[pl.ds(wrow0 + c * _C, _C), :])


def _sc_body(split, seq_hbm, item_hbm, table_hbm, pooled_hbm, item_out_hbm,
             idx_lmaj, rows_a, rows_b, pooled_v, sem_a, sem_b):
    wid = lax.axis_index("s") * _NC + lax.axis_index("c")
    wrow0 = wid * _RPW
    grow0 = split * _NBATCH + wrow0

    stages = [
        pltpu.async_copy(seq_hbm.at[pl.ds(l * B + grow0, _RPW)],
                         idx_lmaj.at[pl.ds(l * _ROWLEN, _RPW)], sem_a)
        for l in range(L)
    ]
    for cp in stages:
        cp.wait()

    _issue_chunk(0, idx_lmaj, table_hbm, rows_a, sem_a)

    def pair_body(p, _):
        ca = 2 * p
        cb = 2 * p + 1
        _issue_chunk(cb, idx_lmaj, table_hbm, rows_b, sem_b)
        _wait_chunk(ca, idx_lmaj, table_hbm, rows_a, sem_a)
        _pool_chunk(ca, wrow0, idx_lmaj, rows_a, pooled_v, pooled_hbm)

        @pl.when(p < _NCHUNK // 2 - 1)
        def _():
            _issue_chunk(ca + 2, idx_lmaj, table_hbm, rows_a, sem_a)

        _wait_chunk(cb, idx_lmaj, table_hbm, rows_b, sem_b)
        _pool_chunk(cb, wrow0, idx_lmaj, rows_b, pooled_v, pooled_hbm)
        return ()

    lax.fori_loop(0, _NCHUNK // 2, pair_body, (), unroll=False)

    pltpu.sync_copy(item_hbm.at[pl.ds(wrow0, _RPW)],
                    idx_lmaj.at[pl.ds(0, _RPW)])
    copies = [
        pltpu.async_copy(table_hbm.at[idx_lmaj.at[pl.ds(_GR * k, _GR)]],
                         rows_a.at[pl.ds(_GR * k, _GR), :], sem_a)
        for k in range(_RPW // _GR)
    ]
    for cp in copies:
        cp.wait()
    pltpu.sync_copy(rows_a.at[pl.ds(0, _RPW), :],
                    item_out_hbm.at[pl.ds(wrow0, _RPW), :])


def _sc_gather_pool(split, seq_t_flat, item_split, table):
    kfn = pl.kernel(
        functools.partial(_sc_body, split),
        out_type=(jax.ShapeDtypeStruct((_NBATCH, D), jnp.float32),
                  jax.ShapeDtypeStruct((_NBATCH, D), jnp.float32)),
        mesh=plsc.VectorSubcoreMesh(core_axis_name="c", subcore_axis_name="s"),
        scratch_types=[
            pltpu.VMEM((L * _ROWLEN + _LANES,), jnp.int32),
            pltpu.VMEM((_IPC, D), jnp.float32),
            pltpu.VMEM((_IPC, D), jnp.float32),
            pltpu.VMEM((_C, D), jnp.float32),
            pltpu.SemaphoreType.DMA,
            pltpu.SemaphoreType.DMA,
        ],
    )
    return kfn(seq_t_flat, item_split, table)


_BM = min(2048, B // _NSPLIT)
_NB = _NBATCH // _BM


def _mlp_body(pooled_ref, item_ref, w1_ref, b1_ref, w2_ref, b2_ref, out_ref):
    x = pooled_ref[...]
    h = jnp.dot(x, w1_ref[...], preferred_element_type=jnp.float32)
    h = jnp.maximum(h + b1_ref[...], 0.0)
    u = jnp.dot(h, w2_ref[...], preferred_element_type=jnp.float32)
    u = jnp.maximum(u + b2_ref[...], 0.0)
    s = jnp.sum(u * item_ref[...], axis=1, keepdims=True)
    out_ref[0] = jax.nn.sigmoid(s)


def _tc_mlp(pooled, item_embed, W1b, b1, W2b, b2):
    out = pl.pallas_call(
        _mlp_body,
        grid=(_NB,),
        in_specs=[
            pl.BlockSpec((_BM, D), lambda i: (i, 0)),
            pl.BlockSpec((_BM, D), lambda i: (i, 0)),
            pl.BlockSpec((D, H), lambda i: (0, 0)),
            pl.BlockSpec((1, H), lambda i: (0, 0)),
            pl.BlockSpec((H, D), lambda i: (0, 0)),
            pl.BlockSpec((1, D), lambda i: (0, 0)),
        ],
        out_specs=pl.BlockSpec((1, _BM, 1), lambda i: (i, 0, 0)),
        out_shape=jax.ShapeDtypeStruct((_NB, _BM, 1), jnp.float32),
    )(pooled, item_embed, W1b, b1.reshape(1, H), W2b, b2.reshape(1, D))
    return out.reshape(_NBATCH, 1)


def kernel(seq_inputs, item_inputs, table, W1, b1, W2, b2):
    seq_t_flat = seq_inputs.T.reshape(L * B).astype(jnp.int32)
    item_flat = item_inputs.reshape(B).astype(jnp.int32)
    W1b = W1
    W2b = W2
    outs = []
    for s in range(_NSPLIT):
        pooled, item_embed = _sc_gather_pool(
            s, seq_t_flat, item_flat[s * _NBATCH:(s + 1) * _NBATCH], table)
        outs.append(_tc_mlp(pooled, item_embed, W1b, b1, W2b, b2))
    return jnp.concatenate(outs, axis=0)
